# Initial kernel scaffold; baseline (speedup 1.0000x reference)
import jax
import jax.numpy as jnp
from jax.experimental import pallas as pl


def kernel(x, k_W1, k_b1, k_gamma, k_beta, k_W2, k_b2, storage, memory_values, in_proj_w, in_proj_b, out_proj_w, out_proj_b, c1_W, c1_b, c2_W, c2_b):
    raise NotImplementedError("write your pallas kernel here")


# speedup vs baseline: 1.5327x; 1.5327x over previous
import functools

import jax
import jax.numpy as jnp
import numpy as np
from jax import lax
from jax.experimental import pallas as pl
from jax.experimental.pallas import tpu as pltpu
from jax.experimental.pallas import tpu_sc as plsc

_B = 1024
_D = 64
_M = 100000
_H = 4
_K = 5
_T = 2048
_NT = (_M + _T - 1) // _T
_CW = 128
_BIG = 2 ** 30


def _gelu_exact(h):
    return 0.5 * h * (1.0 + lax.erf(h * np.float32(0.7071067811865476)))


def _enc_body(x_ref, w1_ref, b1_ref, g_ref, bt_ref, w2_ref, b2_ref,
              eq_ref, qn_ref):
    x = x_ref[...]
    h = jax.lax.dot_general(x, w1_ref[...], (((1,), (1,)), ((), ())),
                            preferred_element_type=jnp.float32) + b1_ref[...]
    mu = jnp.mean(h, axis=-1, keepdims=True)
    var = jnp.mean((h - mu) ** 2, axis=-1, keepdims=True)
    h = (h - mu) / jnp.sqrt(var + 1e-5) * g_ref[...] + bt_ref[...]
    h = _gelu_exact(h)
    eq = jax.lax.dot_general(h, w2_ref[...], (((1,), (1,)), ((), ())),
                             preferred_element_type=jnp.float32) + b2_ref[...]
    eq_ref[...] = eq
    nrm = jnp.sqrt(jnp.sum(eq * eq, axis=-1, keepdims=True))
    qn_ref[...] = eq / jnp.maximum(nrm, 1e-8)


def _encode(x, w1, b1, g, bt, w2, b2):
    return pl.pallas_call(
        _enc_body,
        out_shape=[jax.ShapeDtypeStruct((_B, _D), jnp.float32),
                   jax.ShapeDtypeStruct((_B, _D), jnp.float32)],
    )(x, w1, b1.reshape(1, _D), g.reshape(1, _D), bt.reshape(1, _D),
      w2, b2.reshape(1, _D))


def _knn_body(qn_ref, s_ref, oi_ref, cv_ref, ci_ref):
    i = pl.program_id(0)

    @pl.when(i == 0)
    def _init():
        cv_ref[...] = jnp.full((_B, _CW), -jnp.inf, jnp.float32)
        ci_ref[...] = jnp.zeros((_B, _CW), jnp.int32)

    s = s_ref[...]
    nrm = jnp.sqrt(jnp.sum(s * s, axis=1, keepdims=True))
    sn = s / jnp.maximum(nrm, 1e-8)
    sc = jax.lax.dot_general(qn_ref[...], sn, (((1,), (1,)), ((), ())),
                             preferred_element_type=jnp.float32)
    col = i * _T + jax.lax.broadcasted_iota(jnp.int32, (1, _T), 1)
    sc = jnp.where(col < _M, sc, -jnp.inf)

    A = jnp.concatenate([cv_ref[...], sc], axis=1)
    AI = jnp.concatenate(
        [ci_ref[...], jnp.broadcast_to(col, (_B, _T))], axis=1)
    lane = jax.lax.broadcasted_iota(jnp.int32, (1, _CW + _T), 1)

    nv, ni = [], []
    for _ in range(_K):
        m = jnp.max(A, axis=1, keepdims=True)
        am = jnp.min(jnp.where(A == m, lane, _BIG), axis=1, keepdims=True)
        hit = lane == am
        gi = jnp.min(jnp.where(hit, AI, _BIG), axis=1, keepdims=True)
        nv.append(m)
        ni.append(gi)
        A = jnp.where(hit, -jnp.inf, A)

    lane128 = jax.lax.broadcasted_iota(jnp.int32, (1, _CW), 1)
    cv = jnp.full((_B, _CW), -jnp.inf, jnp.float32)
    ci = jnp.zeros((_B, _CW), jnp.int32)
    for j in range(_K):
        sel = lane128 == j
        cv = jnp.where(sel, nv[j], cv)
        ci = jnp.where(sel, ni[j], ci)
    cv_ref[...] = cv
    ci_ref[...] = ci
    oi_ref[...] = ci


def _knn_topk(qn, storage):
    return pl.pallas_call(
        _knn_body,
        grid=(_NT,),
        in_specs=[
            pl.BlockSpec((_B, _D), lambda i: (0, 0)),
            pl.BlockSpec((_T, _D), lambda i: (i, 0)),
        ],
        out_specs=pl.BlockSpec((_B, _CW), lambda i: (0, 0)),
        out_shape=jax.ShapeDtypeStruct((_B, _CW), jnp.int32),
        scratch_shapes=[
            pltpu.VMEM((_B, _CW), jnp.float32),
            pltpu.VMEM((_B, _CW), jnp.int32),
        ],
        compiler_params=pltpu.CompilerParams(
            dimension_semantics=("arbitrary",)),
    )(qn, storage)


def _gather_rows(table, idx_flat):
    n = idx_flat.shape[0]
    nw = 32
    bpw = n // nw
    mesh = plsc.VectorSubcoreMesh(core_axis_name="c", subcore_axis_name="s")

    @functools.partial(
        pl.kernel,
        mesh=mesh,
        out_type=jax.ShapeDtypeStruct((n, _D), jnp.float32),
        scratch_types=[
            pltpu.VMEM((bpw,), jnp.int32),
            pltpu.VMEM((bpw, _D), jnp.float32),
            pltpu.SemaphoreType.DMA,
        ],
        compiler_params=pltpu.CompilerParams(use_tc_tiling_on_sc=False),
    )
    def k(table_hbm, idx_hbm, out_hbm, idx_v, rows_v, sem):
        wid = lax.axis_index("s") * 2 + lax.axis_index("c")
        base = wid * bpw
        pltpu.sync_copy(idx_hbm.at[pl.ds(base, bpw)], idx_v)
        pltpu.async_copy(table_hbm.at[idx_v], rows_v, sem).wait()
        pltpu.sync_copy(rows_v, out_hbm.at[pl.ds(base, bpw)])

    return k(table, idx_flat)


def _post_body(x_ref, eq_ref, r_ref, wq_ref, bq_ref, wk_ref, bk_ref,
               wv_ref, bv_ref, wo_ref, bo_ref, c1w_ref, c1b_ref,
               c2w_ref, c2b_ref, seg_ref, out_ref):
    eq = eq_ref[...]
    q = jax.lax.dot_general(eq, wq_ref[...], (((1,), (1,)), ((), ())),
                            preferred_element_type=jnp.float32) + bq_ref[...]
    seg = seg_ref[...]
    scs, vs = [], []
    for j in range(_K):
        r = r_ref[:, j * _D:(j + 1) * _D]
        kj = jax.lax.dot_general(r, wk_ref[...], (((1,), (1,)), ((), ())),
                                 preferred_element_type=jnp.float32) + bk_ref[...]
        vj = jax.lax.dot_general(r, wv_ref[...], (((1,), (1,)), ((), ())),
                                 preferred_element_type=jnp.float32) + bv_ref[...]
        sj = jax.lax.dot_general(q * kj, seg, (((1,), (1,)), ((), ())),
                                 preferred_element_type=jnp.float32)
        scs.append(sj * 0.25)
        vs.append(vj)
    m = scs[0]
    for j in range(1, _K):
        m = jnp.maximum(m, scs[j])
    es = [jnp.exp(s - m) for s in scs]
    tot = es[0]
    for j in range(1, _K):
        tot = tot + es[j]
    ctx = jnp.zeros((_B, _D), jnp.float32)
    for j in range(_K):
        a = es[j] / tot
        ab = jax.lax.dot_general(a, seg, (((1,), (0,)), ((), ())),
                                 preferred_element_type=jnp.float32)
        ctx = ctx + ab * vs[j]
    comp = jax.lax.dot_general(ctx, wo_ref[...], (((1,), (1,)), ((), ())),
                               preferred_element_type=jnp.float32) + bo_ref[...]
    h = jax.lax.dot_general(comp, c1w_ref[...], (((1,), (1,)), ((), ())),
                            preferred_element_type=jnp.float32) + c1b_ref[...]
    h = _gelu_exact(h)
    ca1 = jax.lax.dot_general(h, c2w_ref[...], (((1,), (1,)), ((), ())),
                              preferred_element_type=jnp.float32) + c2b_ref[...]
    out_ref[...] = x_ref[...] + 0.5 * ca1


def _post(x, eq, retrieved_flat, in_proj_w, in_proj_b, out_proj_w, out_proj_b,
          c1_W, c1_b, c2_W, c2_b):
    seg = np.zeros((_H, _D), np.float32)
    for h in range(_H):
        seg[h, h * 16:(h + 1) * 16] = 1.0
    seg = jnp.asarray(seg)
    wq, wk, wv = in_proj_w[:_D], in_proj_w[_D:2 * _D], in_proj_w[2 * _D:]
    bq, bk, bv = in_proj_b[:_D], in_proj_b[_D:2 * _D], in_proj_b[2 * _D:]
    return pl.pallas_call(
        _post_body,
        out_shape=jax.ShapeDtypeStruct((_B, _D), jnp.float32),
    )(x, eq, retrieved_flat, wq, bq.reshape(1, _D), wk, bk.reshape(1, _D),
      wv, bv.reshape(1, _D), out_proj_w, out_proj_b.reshape(1, _D),
      c1_W, c1_b.reshape(1, 2 * _D), c2_W, c2_b.reshape(1, _D), seg)


def kernel(x, k_W1, k_b1, k_gamma, k_beta, k_W2, k_b2, storage, memory_values,
           in_proj_w, in_proj_b, out_proj_w, out_proj_b, c1_W, c1_b, c2_W,
           c2_b):
    eq, qn = _encode(x, k_W1, k_b1, k_gamma, k_beta, k_W2, k_b2)
    top = _knn_topk(qn, storage)
    idx_flat = top[:, :_K].reshape(_B * _K)
    retrieved = _gather_rows(memory_values, idx_flat)
    retrieved_flat = retrieved.reshape(_B, _K * _D)
    return _post(x, eq, retrieved_flat, in_proj_w, in_proj_b,
                 out_proj_w, out_proj_b, c1_W, c1_b, c2_W, c2_b)
